# baseline (speedup 1.0000x reference)
import jax
import jax.numpy as jnp
from jax.experimental import pallas as pl
from jax.experimental.pallas import tpu as pltpu

_DIM = 768
_NE = 8
_TOKENS = 32768
_BLK = 4096


def _router_body(x_ref, w_ref, b_ref, out_ref):
    logits = jnp.dot(x_ref[...], w_ref[...]) + b_ref[...]
    mx = jnp.max(logits, axis=1, keepdims=True)
    ids = jax.lax.broadcasted_iota(jnp.int32, (_BLK, _NE), 1)
    idx = jnp.min(jnp.where(logits == mx, ids, _NE), axis=1, keepdims=True)
    ids16 = jax.lax.broadcasted_iota(jnp.int32, (_BLK, 2 * _NE), 1)
    out_ref[...] = jnp.where(
        ids16 < _NE,
        (ids16 == idx).astype(jnp.float32),
        idx.astype(jnp.float32),
    )


def kernel(x, W, b):
    b2 = b.reshape(1, _NE)
    grid = (_TOKENS // _BLK,)
    out = pl.pallas_call(
        _router_body,
        grid=grid,
        in_specs=[
            pl.BlockSpec((_BLK, _DIM), lambda i: (i, 0)),
            pl.BlockSpec((_DIM, _NE), lambda i: (0, 0)),
            pl.BlockSpec((1, _NE), lambda i: (0, 0)),
        ],
        out_specs=pl.BlockSpec((_BLK, 2 * _NE), lambda i: (i, 0)),
        out_shape=jax.ShapeDtypeStruct((_TOKENS, 2 * _NE), jnp.float32),
        compiler_params=pltpu.CompilerParams(
            dimension_semantics=("arbitrary",),
        ),
    )(x, W, b2)
    router = out[:, :_NE]
    idx = out[:, _NE:_NE + 1].astype(jnp.int32)
    return (router, idx)


# speedup vs baseline: 1.0876x; 1.0876x over previous
import jax
import jax.numpy as jnp
from jax.experimental import pallas as pl
from jax.experimental.pallas import tpu as pltpu

_DIM = 768
_NE = 8
_TOKENS = 32768
_BLK = 4096


def _router_body(x_ref, w_ref, b_ref, router_ref, idx_hbm, idx_buf, sem):
    i = pl.program_id(0)
    logits = jnp.dot(x_ref[...], w_ref[...]) + b_ref[...]
    mx = jnp.max(logits, axis=1, keepdims=True)
    ids = jax.lax.broadcasted_iota(jnp.int32, (_BLK, _NE), 1)
    idx = jnp.min(jnp.where(logits == mx, ids, _NE), axis=1, keepdims=True)
    router_ref[...] = (ids == idx).astype(jnp.float32)
    idx_buf[...] = jnp.broadcast_to(idx, (_BLK, _NE))
    cp = pltpu.make_async_copy(
        idx_buf, idx_hbm.at[pl.ds(i * _BLK, _BLK), :], sem)
    cp.start()
    cp.wait()


def kernel(x, W, b):
    b2 = b.reshape(1, _NE)
    grid = (_TOKENS // _BLK,)
    router, idx8 = pl.pallas_call(
        _router_body,
        grid=grid,
        in_specs=[
            pl.BlockSpec((_BLK, _DIM), lambda i: (i, 0)),
            pl.BlockSpec((_DIM, _NE), lambda i: (0, 0)),
            pl.BlockSpec((1, _NE), lambda i: (0, 0)),
        ],
        out_specs=[
            pl.BlockSpec((_BLK, _NE), lambda i: (i, 0)),
            pl.BlockSpec(memory_space=pl.ANY),
        ],
        out_shape=[
            jax.ShapeDtypeStruct((_TOKENS, _NE), jnp.float32),
            jax.ShapeDtypeStruct((_TOKENS, _NE), jnp.int32),
        ],
        scratch_shapes=[
            pltpu.VMEM((_BLK, _NE), jnp.int32),
            pltpu.SemaphoreType.DMA,
        ],
        compiler_params=pltpu.CompilerParams(
            dimension_semantics=("arbitrary",),
        ),
    )(x, W, b2)
    return (router, idx8[:, 0:1])
